# baseline (speedup 1.0000x reference)
import functools

import jax
import jax.numpy as jnp
from jax import lax
from jax.experimental import pallas as pl
from jax.experimental.pallas import tpu as pltpu
from jax.experimental.pallas import tpu_sc as plsc

B, N, M, C = 4, 8192, 2048, 64
NS = 32
R2 = 0.4 * 0.4
BM = B * M
NW = 32
CPW = BM // NW
WPB = M // CPW
RPW = NS * CPW
KCH = RPW // 128


def _sc_fused(xyz_t, new_t, feat_rows):
    mesh = plsc.VectorSubcoreMesh(core_axis_name="c", subcore_axis_name="s")

    @functools.partial(
        pl.kernel,
        out_type=(
            jax.ShapeDtypeStruct((NS * BM, C), jnp.float32),
            jax.ShapeDtypeStruct((NW, CPW * NS), jnp.float32),
            jax.ShapeDtypeStruct((NW, CPW * NS), jnp.float32),
            jax.ShapeDtypeStruct((NW, CPW * NS), jnp.float32),
        ),
        mesh=mesh,
        compiler_params=pltpu.CompilerParams(
            needs_layout_passes=False, use_tc_tiling_on_sc=False),
        scratch_types=[
            pltpu.VMEM((N,), jnp.float32),
            pltpu.VMEM((N,), jnp.float32),
            pltpu.VMEM((N,), jnp.float32),
            pltpu.VMEM((CPW,), jnp.float32),
            pltpu.VMEM((CPW,), jnp.float32),
            pltpu.VMEM((CPW,), jnp.float32),
            pltpu.VMEM((NS * CPW,), jnp.int32),
            pltpu.VMEM((CPW * NS,), jnp.float32),
            pltpu.VMEM((CPW * NS,), jnp.float32),
            pltpu.VMEM((CPW * NS,), jnp.float32),
            pltpu.VMEM((128, C), jnp.float32),
            pltpu.VMEM((128, C), jnp.float32),
            pltpu.SemaphoreType.DMA,
            pltpu.SemaphoreType.DMA,
        ],
    )
    def k(xyz_hbm, new_hbm, feat_hbm, gf_hbm, gxx_hbm, gxy_hbm, gxz_hbm,
          px_v, py_v, pz_v, cx_v, cy_v, cz_v, idx_v, gxx_v, gxy_v, gxz_v,
          rows_a, rows_b, sem_a, sem_b):
        wid = lax.axis_index("c") * 16 + lax.axis_index("s")
        b = wid // WPB
        moff = (wid % WPB) * CPW
        pltpu.sync_copy(xyz_hbm.at[pl.ds((b * 3 + 0) * N, N)], px_v)
        pltpu.sync_copy(xyz_hbm.at[pl.ds((b * 3 + 1) * N, N)], py_v)
        pltpu.sync_copy(xyz_hbm.at[pl.ds((b * 3 + 2) * N, N)], pz_v)
        pltpu.sync_copy(new_hbm.at[pl.ds((b * 3 + 0) * M + moff, CPW)], cx_v)
        pltpu.sync_copy(new_hbm.at[pl.ds((b * 3 + 1) * M + moff, CPW)], cy_v)
        pltpu.sync_copy(new_hbm.at[pl.ds((b * 3 + 2) * M + moff, CPW)], cz_v)

        lanes = lax.iota(jnp.int32, 16)
        zeros16 = jnp.zeros((16,), jnp.int32)
        boff = b * N

        def center_body(m, carry):
            mvec = jnp.full((16,), m, jnp.int32)
            mq = (m // 16) * 16
            ml = jnp.full((16,), lax.rem(m, 16), jnp.int32)
            cx = cx_v[pl.ds(mq, 16)].at[ml].get(mode="promise_in_bounds")
            cy = cy_v[pl.ds(mq, 16)].at[ml].get(mode="promise_in_bounds")
            cz = cz_v[pl.ds(mq, 16)].at[ml].get(mode="promise_in_bounds")

            def cond(st):
                return (st[0] < N // 16) & (st[1] < NS)

            def body(st):
                j, cnt = st
                base = j * 16
                px = px_v[pl.ds(base, 16)]
                py = py_v[pl.ds(base, 16)]
                pz = pz_v[pl.ds(base, 16)]
                dx = px - cx
                dy = py - cy
                dz = pz - cz
                sq = dx * dx + dy * dy + dz * dz
                msk = sq < R2
                pre = plsc.cumsum(msk.astype(jnp.int32))
                pos = cnt + pre - 1
                okm = msk & (pos < NS)
                jv = base + lanes + boff
                plsc.store_scatter(idx_v, [pos * CPW + mvec], jv, mask=okm)
                plsc.store_scatter(gxx_v, [mvec * NS + pos], dx, mask=okm)
                plsc.store_scatter(gxy_v, [mvec * NS + pos], dy, mask=okm)
                plsc.store_scatter(gxz_v, [mvec * NS + pos], dz, mask=okm)
                return j + 1, cnt + pre[15]

            _, cnt = lax.while_loop(cond, body,
                                    (jnp.int32(0), jnp.int32(0)))

            @pl.when(cnt < NS)
            def _fill():
                row = idx_v[pl.ds(m, 16)]
                fgi0 = row.at[zeros16].get(mode="promise_in_bounds")
                fgiv = jnp.where(cnt > 0, fgi0,
                                 jnp.full((16,), boff, jnp.int32))
                fl = fgiv - boff
                fx = plsc.load_gather(px_v, [fl]) - cx
                fy = plsc.load_gather(py_v, [fl]) - cy
                fz = plsc.load_gather(pz_v, [fl]) - cz
                for h in (0, 16):
                    lp = lanes + h
                    fm = lp >= cnt
                    plsc.store_scatter(idx_v, [lp * CPW + mvec], fgiv,
                                       mask=fm)
                    plsc.store_scatter(gxx_v, [mvec * NS + lp], fx, mask=fm)
                    plsc.store_scatter(gxy_v, [mvec * NS + lp], fy, mask=fm)
                    plsc.store_scatter(gxz_v, [mvec * NS + lp], fz, mask=fm)

            return carry

        lax.fori_loop(0, CPW, center_body, 0)
        pltpu.sync_copy(gxx_v, gxx_hbm.at[wid])
        pltpu.sync_copy(gxy_v, gxy_hbm.at[wid])
        pltpu.sync_copy(gxz_v, gxz_hbm.at[wid])

        gbase = wid * RPW
        pltpu.async_copy(feat_hbm.at[idx_v.at[pl.ds(0, 128)]], rows_a, sem_a)

        def gbody(j, _):
            even = lax.rem(j, 2) == 0

            @pl.when((j + 1) < KCH)
            def _prefetch():
                nxt = idx_v.at[pl.ds((j + 1) * 128, 128)]

                @pl.when(even)
                def _():
                    pltpu.async_copy(feat_hbm.at[nxt], rows_b, sem_b)

                @pl.when(jnp.logical_not(even))
                def _():
                    pltpu.async_copy(feat_hbm.at[nxt], rows_a, sem_a)

            cur = idx_v.at[pl.ds(j * 128, 128)]

            @pl.when(even)
            def _drain_a():
                pltpu.make_async_copy(feat_hbm.at[cur], rows_a, sem_a).wait()
                pltpu.sync_copy(rows_a,
                                gf_hbm.at[pl.ds(gbase + j * 128, 128), :])

            @pl.when(jnp.logical_not(even))
            def _drain_b():
                pltpu.make_async_copy(feat_hbm.at[cur], rows_b, sem_b).wait()
                pltpu.sync_copy(rows_b,
                                gf_hbm.at[pl.ds(gbase + j * 128, 128), :])

            return 0

        lax.fori_loop(0, KCH, gbody, 0)

    return k(xyz_t, new_t, feat_rows)


def _tc_mlp(gf4, gxx, gxy, gxz, W1, b1, W2, b2, W3, b3):
    W1xT = W1[:, :3].T
    W1fT = W1[:, 3:].T
    W2T = W2.T
    W3T = W3.T
    b1r = b1.reshape(1, 64)
    b2r = b2.reshape(1, 64)
    b3r = b3.reshape(1, 128)

    def body(gf_ref, gxx_ref, gxy_ref, gxz_ref, w1x_ref, w1f_ref, w2_ref,
             w3_ref, b1_ref, b2_ref, b3_ref, out_ref):
        gx = gxx_ref[0]
        gy = gxy_ref[0]
        gz = gxz_ref[0]
        sq = gx * gx + gy * gy + gz * gz
        dr = 1.0 / (jnp.sqrt(sq) + 1e-8)
        w = dr / jnp.sum(dr, axis=1, keepdims=True)
        w1x = w1x_ref[...]
        w1f = w1f_ref[...]
        w2 = w2_ref[...]
        w3 = w3_ref[...]
        bb1 = b1_ref[...]
        bb2 = b2_ref[...]
        bb3 = b3_ref[...]
        acc = jnp.zeros((CPW, 128), jnp.float32)
        for ns in range(NS):
            xf = gf_ref[0, ns]
            xc = (gx[:, ns:ns + 1] * w1x[0:1, :]
                  + gy[:, ns:ns + 1] * w1x[1:2, :]
                  + gz[:, ns:ns + 1] * w1x[2:3, :])
            h = jnp.maximum(
                jnp.dot(xf, w1f, preferred_element_type=jnp.float32,
                        precision=lax.Precision.HIGHEST) + xc + bb1, 0.0)
            h = jnp.maximum(
                jnp.dot(h, w2, preferred_element_type=jnp.float32,
                        precision=lax.Precision.HIGHEST) + bb2, 0.0)
            h = jnp.maximum(
                jnp.dot(h, w3, preferred_element_type=jnp.float32,
                        precision=lax.Precision.HIGHEST) + bb3, 0.0)
            acc = acc + w[:, ns:ns + 1] * h
        out_ref[...] = acc

    grid = (NW,)
    return pl.pallas_call(
        body,
        grid=grid,
        compiler_params=pltpu.CompilerParams(
            vmem_limit_bytes=100 * 1024 * 1024),
        in_specs=[
            pl.BlockSpec((1, NS, CPW, C), lambda i: (i, 0, 0, 0)),
            pl.BlockSpec((1, CPW, NS), lambda i: (i, 0, 0)),
            pl.BlockSpec((1, CPW, NS), lambda i: (i, 0, 0)),
            pl.BlockSpec((1, CPW, NS), lambda i: (i, 0, 0)),
            pl.BlockSpec((3, 64), lambda i: (0, 0)),
            pl.BlockSpec((64, 64), lambda i: (0, 0)),
            pl.BlockSpec((64, 64), lambda i: (0, 0)),
            pl.BlockSpec((64, 128), lambda i: (0, 0)),
            pl.BlockSpec((1, 64), lambda i: (0, 0)),
            pl.BlockSpec((1, 64), lambda i: (0, 0)),
            pl.BlockSpec((1, 128), lambda i: (0, 0)),
        ],
        out_specs=pl.BlockSpec((CPW, 128), lambda i: (i, 0)),
        out_shape=jax.ShapeDtypeStruct((BM, 128), jnp.float32),
    )(gf4, gxx, gxy, gxz, W1xT, W1fT, W2T, W3T, b1r, b2r, b3r)


def kernel(xyz, new_xyz, features, W1, b1, W2, b2, W3, b3):
    xyz_t = jnp.transpose(xyz, (0, 2, 1)).reshape(-1)
    new_t = jnp.transpose(new_xyz, (0, 2, 1)).reshape(-1)
    feat_rows = jnp.transpose(features, (0, 2, 1)).reshape(B * N, C)
    gf, gxx_raw, gxy_raw, gxz_raw = _sc_fused(xyz_t, new_t, feat_rows)
    out = _tc_mlp(gf.reshape(NW, NS, CPW, C),
                  gxx_raw.reshape(NW, CPW, NS),
                  gxy_raw.reshape(NW, CPW, NS),
                  gxz_raw.reshape(NW, CPW, NS),
                  W1, b1, W2, b2, W3, b3)
    out = out.reshape(B, M, 128).transpose(0, 2, 1)
    return (new_xyz, out)
